# baseline (device time: 953559 ns/iter reference)
import jax
import jax.numpy as jnp
from jax import lax
from jax.experimental import pallas as pl
from jax.experimental.pallas import tpu as pltpu

T = 1024
V_SHARD = 16384


def _stats_exchange(stats):

    def body(stats_ref, out_ref, send_sem, recv_sem):
        my_x = lax.axis_index("x")
        my_y = lax.axis_index("y")
        peer = (my_x, 1 - my_y)

        barrier = pltpu.get_barrier_semaphore()
        pl.semaphore_signal(barrier, inc=1, device_id=peer,
                            device_id_type=pl.DeviceIdType.MESH)
        pl.semaphore_wait(barrier, 1)

        rdma = pltpu.make_async_remote_copy(
            src_ref=stats_ref,
            dst_ref=out_ref,
            send_sem=send_sem,
            recv_sem=recv_sem,
            device_id=peer,
            device_id_type=pl.DeviceIdType.MESH,
        )
        rdma.start()
        rdma.wait()

    return pl.pallas_call(
        body,
        out_shape=jax.ShapeDtypeStruct(stats.shape, stats.dtype),
        in_specs=[pl.BlockSpec(memory_space=pltpu.VMEM)],
        out_specs=pl.BlockSpec(memory_space=pltpu.VMEM),
        scratch_shapes=[pltpu.SemaphoreType.DMA, pltpu.SemaphoreType.DMA],
        compiler_params=pltpu.CompilerParams(collective_id=0),
    )(stats)


NCH = 16
CH = T // NCH
NS = 2
NR = 4


def _halves_exchange(logits, mvec, svec):

    def body(logits_ref, mvec_ref, svec_ref, out_ref, sbuf, rbuf,
             stage_sems, loc_sem, out_sems, send_sems, recv_sems,
             credit_sem):
        my_x = lax.axis_index("x")
        my_y = lax.axis_index("y")
        peer = (my_x, 1 - my_y)

        barrier = pltpu.get_barrier_semaphore()
        pl.semaphore_signal(barrier, inc=1, device_id=peer,
                            device_id_type=pl.DeviceIdType.MESH)
        pl.semaphore_wait(barrier, 1)

        sends = []
        outcps = []

        def recv_one(rc):
            slot = rc % NR
            rd = pltpu.make_async_remote_copy(
                src_ref=sbuf.at[0],
                dst_ref=rbuf.at[slot],
                send_sem=stage_sems.at[0],
                recv_sem=recv_sems.at[slot],
                device_id=peer, device_id_type=pl.DeviceIdType.MESH)
            rd.wait_recv()
            cp = pltpu.make_async_copy(
                rbuf.at[slot],
                out_ref.at[pl.ds(rc * CH, CH),
                           pl.ds((1 - my_y) * V_SHARD, V_SHARD)],
                loc_sem)
            cp.start()
            cp.wait()
            pl.semaphore_signal(credit_sem, inc=1, device_id=peer,
                                device_id_type=pl.DeviceIdType.MESH)

        for c in range(NCH):
            sslot = c % NS
            if c >= NS:
                sends[c - NS].wait_send()
                outcps[c - NS].wait()
            st = pltpu.make_async_copy(
                logits_ref.at[pl.ds(c * CH, CH)], sbuf.at[sslot],
                stage_sems.at[sslot])
            st.start()
            st.wait()
            mm = mvec_ref[pl.ds(c * CH, CH), :]
            ss = svec_ref[pl.ds(c * CH, CH), :]
            sbuf[sslot] = jnp.exp(sbuf[sslot] - mm) * ss
            if c >= NR:
                pl.semaphore_wait(credit_sem, 1)
            rd = pltpu.make_async_remote_copy(
                src_ref=sbuf.at[sslot],
                dst_ref=rbuf.at[c % NR],
                send_sem=send_sems.at[sslot],
                recv_sem=recv_sems.at[c % NR],
                device_id=peer, device_id_type=pl.DeviceIdType.MESH)
            rd.start()
            sends.append(rd)
            oc = pltpu.make_async_copy(
                sbuf.at[sslot],
                out_ref.at[pl.ds(c * CH, CH),
                           pl.ds(my_y * V_SHARD, V_SHARD)],
                out_sems.at[sslot])
            oc.start()
            outcps.append(oc)
            if c >= NR - 1:
                recv_one(c - (NR - 1))
        for rc in range(NCH - (NR - 1), NCH):
            recv_one(rc)
        for c in range(NCH - NS, NCH):
            sends[c].wait_send()
            outcps[c].wait()
        pl.semaphore_wait(credit_sem, NR)

    return pl.pallas_call(
        body,
        out_shape=jax.ShapeDtypeStruct((T, 2 * V_SHARD), logits.dtype),
        in_specs=[pl.BlockSpec(memory_space=pl.ANY),
                  pl.BlockSpec(memory_space=pltpu.VMEM),
                  pl.BlockSpec(memory_space=pltpu.VMEM)],
        out_specs=pl.BlockSpec(memory_space=pl.ANY),
        scratch_shapes=[
            pltpu.VMEM((NS, CH, V_SHARD), jnp.float32),
            pltpu.VMEM((NR, CH, V_SHARD), jnp.float32),
            pltpu.SemaphoreType.DMA((NS,)),
            pltpu.SemaphoreType.DMA,
            pltpu.SemaphoreType.DMA((NS,)),
            pltpu.SemaphoreType.DMA((NS,)),
            pltpu.SemaphoreType.DMA((NR,)),
            pltpu.SemaphoreType.REGULAR,
        ],
        compiler_params=pltpu.CompilerParams(
            collective_id=1, vmem_limit_bytes=100 * 1024 * 1024),
    )(logits, mvec, svec)


def kernel(x, W):
    logits = jax.lax.dot(x.astype(jnp.bfloat16), W.astype(jnp.bfloat16),
                         preferred_element_type=jnp.float32)
    m = logits.max(axis=-1, keepdims=True)
    s = jnp.sum(jnp.exp(logits - m), axis=-1, keepdims=True)
    stats = jnp.concatenate([m, s], axis=-1)

    rem = _stats_exchange(stats)
    m_r, s_r = rem[:, 0:1], rem[:, 1:2]
    M = jnp.maximum(m, m_r)
    S = jnp.exp(m - M) * s + jnp.exp(m_r - M) * s_r
    return _halves_exchange(logits, M, 1.0 / S)


# device time: 939899 ns/iter; 1.0145x vs baseline; 1.0145x over previous
import jax
import jax.numpy as jnp
from jax import lax
from jax.experimental import pallas as pl
from jax.experimental.pallas import tpu as pltpu

T = 1024
V_SHARD = 16384


def _stats_exchange(stats):

    def body(stats_ref, out_ref, send_sem, recv_sem):
        my_x = lax.axis_index("x")
        my_y = lax.axis_index("y")
        peer = (my_x, 1 - my_y)

        barrier = pltpu.get_barrier_semaphore()
        pl.semaphore_signal(barrier, inc=1, device_id=peer,
                            device_id_type=pl.DeviceIdType.MESH)
        pl.semaphore_wait(barrier, 1)

        rdma = pltpu.make_async_remote_copy(
            src_ref=stats_ref,
            dst_ref=out_ref,
            send_sem=send_sem,
            recv_sem=recv_sem,
            device_id=peer,
            device_id_type=pl.DeviceIdType.MESH,
        )
        rdma.start()
        rdma.wait()

    return pl.pallas_call(
        body,
        out_shape=jax.ShapeDtypeStruct(stats.shape, stats.dtype),
        in_specs=[pl.BlockSpec(memory_space=pltpu.VMEM)],
        out_specs=pl.BlockSpec(memory_space=pltpu.VMEM),
        scratch_shapes=[pltpu.SemaphoreType.DMA, pltpu.SemaphoreType.DMA],
        compiler_params=pltpu.CompilerParams(collective_id=0),
    )(stats)


NCH = 16
CH = T // NCH
NS = 2
NR = 4


def _halves_exchange(logits, mvec, svec):

    def body(logits_ref, mvec_ref, svec_ref, out_ref, sbuf, rbuf,
             stage_sems, loc_sem, out_sems, send_sems, recv_sems,
             credit_sem):
        my_x = lax.axis_index("x")
        my_y = lax.axis_index("y")
        peer = (my_x, 1 - my_y)

        barrier = pltpu.get_barrier_semaphore()
        pl.semaphore_signal(barrier, inc=1, device_id=peer,
                            device_id_type=pl.DeviceIdType.MESH)
        pl.semaphore_wait(barrier, 1)

        sends = []
        outcps = []

        def recv_one(rc):
            slot = rc % NR
            rd = pltpu.make_async_remote_copy(
                src_ref=sbuf.at[0],
                dst_ref=rbuf.at[slot],
                send_sem=stage_sems.at[0],
                recv_sem=recv_sems.at[slot],
                device_id=peer, device_id_type=pl.DeviceIdType.MESH)
            rd.wait_recv()
            cp = pltpu.make_async_copy(
                rbuf.at[slot],
                out_ref.at[pl.ds(rc * CH, CH),
                           pl.ds((1 - my_y) * V_SHARD, V_SHARD)],
                loc_sem)
            cp.start()
            cp.wait()
            pl.semaphore_signal(credit_sem, inc=1, device_id=peer,
                                device_id_type=pl.DeviceIdType.MESH)

        for c in range(NCH):
            sslot = c % NS
            if c >= NS:
                sends[c - NS].wait_send()
                outcps[c - NS].wait()
            st = pltpu.make_async_copy(
                logits_ref.at[pl.ds(c * CH, CH)], sbuf.at[sslot],
                stage_sems.at[sslot])
            st.start()
            st.wait()
            mm = mvec_ref[pl.ds(c * CH, CH), :]
            ss = svec_ref[pl.ds(c * CH, CH), :]
            sbuf[sslot] = jnp.exp(sbuf[sslot] - mm) * ss
            if c >= NR:
                pl.semaphore_wait(credit_sem, 1)
            rd = pltpu.make_async_remote_copy(
                src_ref=sbuf.at[sslot],
                dst_ref=rbuf.at[c % NR],
                send_sem=send_sems.at[sslot],
                recv_sem=recv_sems.at[c % NR],
                device_id=peer, device_id_type=pl.DeviceIdType.MESH)
            rd.start()
            sends.append(rd)
            oc = pltpu.make_async_copy(
                sbuf.at[sslot],
                out_ref.at[pl.ds(c * CH, CH),
                           pl.ds(my_y * V_SHARD, V_SHARD)],
                out_sems.at[sslot])
            oc.start()
            outcps.append(oc)
            if c >= NR - 1:
                recv_one(c - (NR - 1))
        for rc in range(NCH - (NR - 1), NCH):
            recv_one(rc)
        for c in range(NCH - NS, NCH):
            sends[c].wait_send()
            outcps[c].wait()
        pl.semaphore_wait(credit_sem, NR)

    return pl.pallas_call(
        body,
        out_shape=jax.ShapeDtypeStruct((T, 2 * V_SHARD), logits.dtype),
        in_specs=[pl.BlockSpec(memory_space=pl.ANY),
                  pl.BlockSpec(memory_space=pltpu.VMEM),
                  pl.BlockSpec(memory_space=pltpu.VMEM)],
        out_specs=pl.BlockSpec(memory_space=pl.ANY),
        scratch_shapes=[
            pltpu.VMEM((NS, CH, V_SHARD), jnp.float32),
            pltpu.VMEM((NR, CH, V_SHARD), jnp.float32),
            pltpu.SemaphoreType.DMA((NS,)),
            pltpu.SemaphoreType.DMA,
            pltpu.SemaphoreType.DMA((NS,)),
            pltpu.SemaphoreType.DMA((NS,)),
            pltpu.SemaphoreType.DMA((NR,)),
            pltpu.SemaphoreType.REGULAR,
        ],
        compiler_params=pltpu.CompilerParams(
            collective_id=1, vmem_limit_bytes=100 * 1024 * 1024),
    )(logits, mvec, svec)


def kernel(x, W):
    logits = x @ W
    m = logits.max(axis=-1, keepdims=True)
    s = jnp.sum(jnp.exp(logits - m), axis=-1, keepdims=True)
    stats = jnp.concatenate([m, s], axis=-1)

    rem = _stats_exchange(stats)
    m_r, s_r = rem[:, 0:1], rem[:, 1:2]
    M = jnp.maximum(m, m_r)
    S = jnp.exp(m - M) * s + jnp.exp(m_r - M) * s_r
    return _halves_exchange(logits, M, 1.0 / S)
